# baseline (speedup 1.0000x reference)
import functools

import jax
import jax.numpy as jnp
from jax import lax
from jax.experimental import pallas as pl
from jax.experimental.pallas import tpu as pltpu
from jax.experimental.pallas import tpu_sc as plsc

N, C, H, W = 4, 256, 64, 64
P = H * W
PPAD = P + 2 * W
A = 9
BR = 2048
R = P // BR
TC = 512

NWORK = 32
CHUNKS = 8
PW = P // CHUNKS
FW = PW * A
GL = 16


def _rpn_kernel(x_ref, w_ref, b_ref, dw_ref, db_ref, o_ref, xcat_ref):
    r = pl.program_id(1)

    @pl.when(r == 0)
    def _build_layout():
        zrow = jnp.zeros((W, 3 * C), dtype=jnp.bfloat16)
        xcat_ref[pl.ds(0, W), :] = zrow
        xcat_ref[pl.ds(W + P, W), :] = zrow
        for j in range(P // TC):
            chunk = x_ref[0, :, pl.ds(j * TC, TC)].astype(jnp.bfloat16)
            xcat_ref[pl.ds(W + j * TC, TC), C:2 * C] = chunk.T
        xc = xcat_ref[:, C:2 * C]
        col = jax.lax.broadcasted_iota(jnp.int32, (PPAD, C), 0) % W
        zero = jnp.zeros((), jnp.bfloat16)
        xr = pltpu.roll(xc, 1, 0)
        xcat_ref[:, 0:C] = jnp.where(col != 0, xr, zero)
        xl = pltpu.roll(xc, PPAD - 1, 0)
        xcat_ref[:, 2 * C:3 * C] = jnp.where(col != (W - 1), xl, zero)

    acc = jnp.zeros((BR, C), dtype=jnp.float32)
    base = W + r * BR
    for dy in (-1, 0, 1):
        blk = xcat_ref[pl.ds(base + dy * W, BR), :]
        acc += jnp.dot(blk, w_ref[dy + 1],
                       preferred_element_type=jnp.float32)
    t = jax.nn.relu(acc + b_ref[0]).astype(jnp.bfloat16)
    st = jax.lax.dot_general(dw_ref[...], t, (((0,), (1,)), ((), ())),
                             preferred_element_type=jnp.float32)
    o_ref[0] = jax.nn.sigmoid(st + db_ref[...])


@functools.partial(
    pl.kernel,
    mesh=plsc.VectorSubcoreMesh(core_axis_name="c", subcore_axis_name="s"),
    out_type=jax.ShapeDtypeStruct((N * P * A,), jnp.float32),
    scratch_types=[
        pltpu.VMEM((FW,), jnp.float32),
        pltpu.VMEM((FW,), jnp.float32),
    ],
    compiler_params=pltpu.CompilerParams(needs_layout_passes=False),
)
def _interleave(fgt_hbm, out_hbm, src_v, out_v):
    wid = lax.axis_index("s") * 2 + lax.axis_index("c")
    nn = wid // CHUNKS
    k = wid % CHUNKS
    for aa in range(A):
        pltpu.sync_copy(fgt_hbm.at[pl.ds(nn * (A * P) + aa * P + k * PW, PW)],
                        src_v.at[pl.ds(aa * PW, PW)])
    for cc in range(FW // GL):
        f = lax.iota(jnp.int32, GL) + (cc * GL)
        idx = lax.rem(f, A) * PW + lax.div(f, A)
        out_v[pl.ds(cc * GL, GL)] = plsc.load_gather(src_v, [idx])
    pltpu.sync_copy(out_v, out_hbm.at[pl.ds(nn * (A * P) + k * FW, FW)])


@functools.partial(jax.jit, static_argnames=())
def kernel(x, img_shape, conv1_w, conv1_b, score_w, score_b, offset_w, offset_b):
    n = x.shape[0]
    xf = x.reshape(n, C, P)
    wr = jnp.transpose(conv1_w, (2, 3, 1, 0)).reshape(3, 3 * C, C)
    wr = wr.astype(jnp.bfloat16)
    b2 = conv1_b.reshape(1, C)
    sw = score_w[:, :, 0, 0]
    dw = (sw[1::2] - sw[0::2]).T.astype(jnp.bfloat16)
    db = (score_b[1::2] - score_b[0::2]).reshape(A, 1)

    fgt = pl.pallas_call(
        _rpn_kernel,
        grid=(n, R),
        in_specs=[
            pl.BlockSpec((1, C, P), lambda i, r: (i, 0, 0)),
            pl.BlockSpec((3, 3 * C, C), lambda i, r: (0, 0, 0)),
            pl.BlockSpec((1, C), lambda i, r: (0, 0)),
            pl.BlockSpec((C, A), lambda i, r: (0, 0)),
            pl.BlockSpec((A, 1), lambda i, r: (0, 0)),
        ],
        out_specs=pl.BlockSpec((1, A, BR), lambda i, r: (i, 0, r)),
        out_shape=jax.ShapeDtypeStruct((n, A, P), jnp.float32),
        scratch_shapes=[
            pltpu.VMEM((PPAD, 3 * C), jnp.bfloat16),
        ],
    )(xf, wr, b2, dw, db)

    flat = _interleave(fgt.reshape(n * A * P))
    return flat.reshape(n, P * A // 2, 2)


# speedup vs baseline: 1.0028x; 1.0028x over previous
import functools

import jax
import jax.numpy as jnp
from jax import lax
from jax.experimental import pallas as pl
from jax.experimental.pallas import tpu as pltpu
from jax.experimental.pallas import tpu_sc as plsc

N, C, H, W = 4, 256, 64, 64
P = H * W
PPAD = P + 2 * W
A = 9
BR = 2048
R = P // BR
TC = 512

NWORK = 32
CHUNKS = 8
PW = P // CHUNKS
FW = PW * A
GL = 16


def _rpn_kernel(x_ref, w_ref, b_ref, dw_ref, db_ref, o_ref, xcat_ref):
    r = pl.program_id(1)

    @pl.when(r == 0)
    def _build_layout():
        zrow = jnp.zeros((W, 3 * C), dtype=jnp.bfloat16)
        xcat_ref[pl.ds(0, W), :] = zrow
        xcat_ref[pl.ds(W + P, W), :] = zrow
        for j in range(P // TC):
            chunk = x_ref[0, :, pl.ds(j * TC, TC)].astype(jnp.bfloat16)
            xcat_ref[pl.ds(W + j * TC, TC), C:2 * C] = chunk.T
        xc = xcat_ref[:, C:2 * C]
        col = jax.lax.broadcasted_iota(jnp.int32, (PPAD, C), 0) % W
        zero = jnp.zeros((), jnp.bfloat16)
        xr = pltpu.roll(xc, 1, 0)
        xcat_ref[:, 0:C] = jnp.where(col != 0, xr, zero)
        xl = pltpu.roll(xc, PPAD - 1, 0)
        xcat_ref[:, 2 * C:3 * C] = jnp.where(col != (W - 1), xl, zero)

    acc = jnp.zeros((BR, C), dtype=jnp.float32)
    base = W + r * BR
    for dy in (-1, 0, 1):
        blk = xcat_ref[pl.ds(base + dy * W, BR), :]
        acc += jnp.dot(blk, w_ref[dy + 1],
                       preferred_element_type=jnp.float32)
    t = jax.nn.relu(acc + b_ref[0]).astype(jnp.bfloat16)
    st = jax.lax.dot_general(dw_ref[...], t, (((0,), (1,)), ((), ())),
                             preferred_element_type=jnp.float32)
    o_ref[0] = jax.nn.sigmoid(st + db_ref[...])


@functools.partial(
    pl.kernel,
    mesh=plsc.VectorSubcoreMesh(core_axis_name="c", subcore_axis_name="s"),
    out_type=jax.ShapeDtypeStruct((N * P * A,), jnp.float32),
    scratch_types=[
        pltpu.VMEM((FW,), jnp.float32),
        pltpu.VMEM((FW,), jnp.float32),
        pltpu.SemaphoreType.DMA,
    ],
    compiler_params=pltpu.CompilerParams(needs_layout_passes=False),
)
def _interleave(fgt_hbm, out_hbm, src_v, out_v, sem):
    wid = lax.axis_index("s") * 2 + lax.axis_index("c")
    nn = wid // CHUNKS
    k = wid % CHUNKS
    copies = [
        pltpu.async_copy(fgt_hbm.at[pl.ds(nn * (A * P) + aa * P + k * PW, PW)],
                         src_v.at[pl.ds(aa * PW, PW)], sem)
        for aa in range(A)
    ]
    for c in copies:
        c.wait()
    for cc in range(FW // GL):
        f = lax.iota(jnp.int32, GL) + (cc * GL)
        idx = lax.rem(f, A) * PW + lax.div(f, A)
        out_v[pl.ds(cc * GL, GL)] = plsc.load_gather(src_v, [idx])
    pltpu.sync_copy(out_v, out_hbm.at[pl.ds(nn * (A * P) + k * FW, FW)])


@functools.partial(jax.jit, static_argnames=())
def kernel(x, img_shape, conv1_w, conv1_b, score_w, score_b, offset_w, offset_b):
    n = x.shape[0]
    xf = x.reshape(n, C, P)
    wr = jnp.transpose(conv1_w, (2, 3, 1, 0)).reshape(3, 3 * C, C)
    wr = wr.astype(jnp.bfloat16)
    b2 = conv1_b.reshape(1, C)
    sw = score_w[:, :, 0, 0]
    dw = (sw[1::2] - sw[0::2]).T.astype(jnp.bfloat16)
    db = (score_b[1::2] - score_b[0::2]).reshape(A, 1)

    fgt = pl.pallas_call(
        _rpn_kernel,
        grid=(n, R),
        in_specs=[
            pl.BlockSpec((1, C, P), lambda i, r: (i, 0, 0)),
            pl.BlockSpec((3, 3 * C, C), lambda i, r: (0, 0, 0)),
            pl.BlockSpec((1, C), lambda i, r: (0, 0)),
            pl.BlockSpec((C, A), lambda i, r: (0, 0)),
            pl.BlockSpec((A, 1), lambda i, r: (0, 0)),
        ],
        out_specs=pl.BlockSpec((1, A, BR), lambda i, r: (i, 0, r)),
        out_shape=jax.ShapeDtypeStruct((n, A, P), jnp.float32),
        scratch_shapes=[
            pltpu.VMEM((PPAD, 3 * C), jnp.bfloat16),
        ],
    )(xf, wr, b2, dw, db)

    flat = _interleave(fgt.reshape(n * A * P))
    return flat.reshape(n, P * A // 2, 2)


# speedup vs baseline: 1.7421x; 1.7372x over previous
import functools

import jax
import jax.numpy as jnp
from jax import lax
from jax.experimental import pallas as pl
from jax.experimental.pallas import tpu as pltpu

N, C, H, W = 4, 256, 64, 64
P = H * W
PPAD = P + 2 * W
A = 9
BR = 2048
R = P // BR
TC = 512
IL = 16

NWORK = 32
CHUNKS = 8
PW = P // CHUNKS
FW = PW * A
GL = 16


def _rpn_kernel(x_ref, w_ref, b_ref, dw_ref, db_ref, o_ref, xcat_ref, s_ref):
    r = pl.program_id(1)

    @pl.when(r == 0)
    def _build_layout():
        zrow = jnp.zeros((W, 3 * C), dtype=jnp.bfloat16)
        xcat_ref[pl.ds(0, W), :] = zrow
        xcat_ref[pl.ds(W + P, W), :] = zrow
        for j in range(P // TC):
            chunk = x_ref[0, :, pl.ds(j * TC, TC)].astype(jnp.bfloat16)
            xcat_ref[pl.ds(W + j * TC, TC), C:2 * C] = chunk.T
        xc = xcat_ref[:, C:2 * C]
        col = jax.lax.broadcasted_iota(jnp.int32, (PPAD, C), 0) % W
        zero = jnp.zeros((), jnp.bfloat16)
        xr = pltpu.roll(xc, 1, 0)
        xcat_ref[:, 0:C] = jnp.where(col != 0, xr, zero)
        xl = pltpu.roll(xc, PPAD - 1, 0)
        xcat_ref[:, 2 * C:3 * C] = jnp.where(col != (W - 1), xl, zero)

    acc = jnp.zeros((BR, C), dtype=jnp.float32)
    base = W + r * BR
    for dy in (-1, 0, 1):
        blk = xcat_ref[pl.ds(base + dy * W, BR), :]
        acc += jnp.dot(blk, w_ref[dy + 1],
                       preferred_element_type=jnp.float32)
    t = jax.nn.relu(acc + b_ref[0]).astype(jnp.bfloat16)
    s = jnp.dot(t, dw_ref[...], preferred_element_type=jnp.float32) + db_ref[0]
    s_ref[...] = jax.nn.sigmoid(s)
    pieces = [s_ref[pl.Slice(j, BR // IL, IL), :] for j in range(IL)]
    o_ref[0] = jnp.concatenate(pieces, axis=1)


@functools.partial(jax.jit, static_argnames=())
def kernel(x, img_shape, conv1_w, conv1_b, score_w, score_b, offset_w, offset_b):
    n = x.shape[0]
    xf = x.reshape(n, C, P)
    wr = jnp.transpose(conv1_w, (2, 3, 1, 0)).reshape(3, 3 * C, C)
    wr = wr.astype(jnp.bfloat16)
    b2 = conv1_b.reshape(1, C)
    sw = score_w[:, :, 0, 0]
    dw = (sw[1::2] - sw[0::2]).T.astype(jnp.bfloat16)
    db = (score_b[1::2] - score_b[0::2]).reshape(1, A)

    fgt = pl.pallas_call(
        _rpn_kernel,
        grid=(n, R),
        in_specs=[
            pl.BlockSpec((1, C, P), lambda i, r: (i, 0, 0)),
            pl.BlockSpec((3, 3 * C, C), lambda i, r: (0, 0, 0)),
            pl.BlockSpec((1, C), lambda i, r: (0, 0)),
            pl.BlockSpec((C, A), lambda i, r: (0, 0)),
            pl.BlockSpec((1, A), lambda i, r: (0, 0)),
        ],
        out_specs=pl.BlockSpec((1, BR // IL, A * IL), lambda i, r: (i, r, 0)),
        out_shape=jax.ShapeDtypeStruct((n, P // IL, A * IL), jnp.float32),
        scratch_shapes=[
            pltpu.VMEM((PPAD, 3 * C), jnp.bfloat16),
            pltpu.VMEM((BR, A), jnp.float32),
        ],
    )(xf, wr, b2, dw, db)

    return fgt.reshape(n, P * A // 2, 2)


# speedup vs baseline: 1.8847x; 1.0818x over previous
import functools

import jax
import jax.numpy as jnp
from jax import lax
from jax.experimental import pallas as pl
from jax.experimental.pallas import tpu as pltpu

N, C, H, W = 4, 256, 64, 64
P = H * W
PPAD = P + 2 * W
A = 9
BR = 4096
R = P // BR
TC = 512
IL = 16

NWORK = 32
CHUNKS = 8
PW = P // CHUNKS
FW = PW * A
GL = 16


def _rpn_kernel(x_ref, w_ref, b_ref, dw_ref, db_ref, o_ref, xcat_ref, s_ref):
    r = pl.program_id(1)

    @pl.when(r == 0)
    def _build_layout():
        zrow = jnp.zeros((W, 3 * C), dtype=jnp.bfloat16)
        xcat_ref[pl.ds(0, W), :] = zrow
        xcat_ref[pl.ds(W + P, W), :] = zrow
        for j in range(P // TC):
            chunk = x_ref[0, :, pl.ds(j * TC, TC)].astype(jnp.bfloat16)
            xcat_ref[pl.ds(W + j * TC, TC), C:2 * C] = chunk.T
        xc = xcat_ref[:, C:2 * C]
        col = jax.lax.broadcasted_iota(jnp.int32, (PPAD, C), 0) % W
        zero = jnp.zeros((), jnp.bfloat16)
        xr = pltpu.roll(xc, 1, 0)
        xcat_ref[:, 0:C] = jnp.where(col != 0, xr, zero)
        xl = pltpu.roll(xc, PPAD - 1, 0)
        xcat_ref[:, 2 * C:3 * C] = jnp.where(col != (W - 1), xl, zero)

    acc = jnp.zeros((BR, C), dtype=jnp.float32)
    base = W + r * BR
    for dy in (-1, 0, 1):
        blk = xcat_ref[pl.ds(base + dy * W, BR), :]
        acc += jnp.dot(blk, w_ref[dy + 1],
                       preferred_element_type=jnp.float32)
    t = jax.nn.relu(acc + b_ref[0]).astype(jnp.bfloat16)
    s = jnp.dot(t, dw_ref[...], preferred_element_type=jnp.float32) + db_ref[0]
    s_ref[...] = jax.nn.sigmoid(s)
    pieces = [s_ref[pl.Slice(j, BR // IL, IL), :] for j in range(IL)]
    o_ref[0] = jnp.concatenate(pieces, axis=1)


@functools.partial(jax.jit, static_argnames=())
def kernel(x, img_shape, conv1_w, conv1_b, score_w, score_b, offset_w, offset_b):
    n = x.shape[0]
    xf = x.reshape(n, C, P)
    wr = jnp.transpose(conv1_w, (2, 3, 1, 0)).reshape(3, 3 * C, C)
    wr = wr.astype(jnp.bfloat16)
    b2 = conv1_b.reshape(1, C)
    sw = score_w[:, :, 0, 0]
    dw = (sw[1::2] - sw[0::2]).T.astype(jnp.bfloat16)
    db = (score_b[1::2] - score_b[0::2]).reshape(1, A)

    fgt = pl.pallas_call(
        _rpn_kernel,
        grid=(n, R),
        in_specs=[
            pl.BlockSpec((1, C, P), lambda i, r: (i, 0, 0)),
            pl.BlockSpec((3, 3 * C, C), lambda i, r: (0, 0, 0)),
            pl.BlockSpec((1, C), lambda i, r: (0, 0)),
            pl.BlockSpec((C, A), lambda i, r: (0, 0)),
            pl.BlockSpec((1, A), lambda i, r: (0, 0)),
        ],
        out_specs=pl.BlockSpec((1, BR // IL, A * IL), lambda i, r: (i, r, 0)),
        out_shape=jax.ShapeDtypeStruct((n, P // IL, A * IL), jnp.float32),
        scratch_shapes=[
            pltpu.VMEM((PPAD, 3 * C), jnp.bfloat16),
            pltpu.VMEM((BR, A), jnp.float32),
        ],
    )(xf, wr, b2, dw, db)

    return fgt.reshape(n, P * A // 2, 2)
